# Initial kernel scaffold; baseline (speedup 1.0000x reference)
import jax
import jax.numpy as jnp
from jax.experimental import pallas as pl


def kernel(x, indices):
    raise NotImplementedError("write your pallas kernel here")


# speedup vs baseline: 845.8744x; 845.8744x over previous
import functools

import jax
import jax.numpy as jnp
from jax import lax
from jax.experimental import pallas as pl
from jax.experimental.pallas import tpu as pltpu
from jax.experimental.pallas import tpu_sc as plsc

_P = 2


def _make_unpool(rows, m):
    n = _P * m
    info = plsc.get_sparse_core_info()
    nc, ns, lanes = info.num_cores, info.num_subcores, info.num_lanes
    nw = nc * ns
    rows_per_w = rows // nw

    @functools.partial(
        pl.kernel,
        out_type=jax.ShapeDtypeStruct((rows, 2 * n), jnp.float32),
        mesh=plsc.VectorSubcoreMesh(core_axis_name="c", subcore_axis_name="s"),
        compiler_params=pltpu.CompilerParams(needs_layout_passes=False),
        scratch_types=[
            pltpu.VMEM((2 * m,), jnp.float32),
            pltpu.VMEM((m + lanes,), jnp.int32),
            pltpu.VMEM((2 * n,), jnp.float32),
        ],
    )
    def k(x_hbm, idx_hbm, out_hbm, x_v, idx_v, out_v):
        wid = lax.axis_index("s") * nc + lax.axis_index("c")
        iota = lax.iota(jnp.int32, lanes)
        zeros = jnp.zeros((lanes,), jnp.float32)
        idx_v[pl.ds(m, lanes)] = jnp.full((lanes,), 2 ** 30, jnp.int32)

        def row_body(r, carry):
            row = wid * rows_per_w + r
            pltpu.sync_copy(x_hbm.at[row], x_v)
            pltpu.sync_copy(idx_hbm.at[row], idx_v.at[pl.ds(0, m)])

            def zero_body(i, c):
                out_v[pl.ds(pl.multiple_of(i * lanes, lanes), lanes)] = zeros
                return c

            lax.fori_loop(0, (2 * n) // lanes, zero_body, 0)

            def scat_body(i, c):
                base = i * lanes
                d = plsc.load_gather(idx_v, [base + iota])
                d_next = plsc.load_gather(idx_v, [base + 1 + iota])
                keep = d != d_next
                reals = plsc.load_gather(x_v, [2 * base + 2 * iota])
                imags = plsc.load_gather(x_v, [2 * base + 2 * iota + 1])
                plsc.store_scatter(out_v, [2 * d], reals, mask=keep)
                plsc.store_scatter(out_v, [2 * d + 1], imags, mask=keep)
                return c

            lax.fori_loop(0, m // lanes, scat_body, 0)
            pltpu.sync_copy(out_v, out_hbm.at[row])
            return carry

        lax.fori_loop(0, rows_per_w, row_body, 0)

    return k


def kernel(x, indices):
    B, C, m, _ = x.shape
    rows = B * C
    n = _P * m
    xf = x.reshape(rows, 2 * m)
    idx = indices.reshape(rows, m).astype(jnp.int32)
    out = _make_unpool(rows, m)(xf, idx)
    return out.reshape(B, C, n, 2)


# speedup vs baseline: 2798.1473x; 3.3080x over previous
import functools

import jax
import jax.numpy as jnp
from jax import lax
from jax.experimental import pallas as pl
from jax.experimental.pallas import tpu as pltpu
from jax.experimental.pallas import tpu_sc as plsc

_P = 2


def _make_unpool(B, C, m):
    n = _P * m
    rows = B * C
    info = plsc.get_sparse_core_info()
    nc, ns, lanes = info.num_cores, info.num_subcores, info.num_lanes
    nw = nc * ns
    rows_per_w = rows // nw

    @functools.partial(
        pl.kernel,
        out_type=jax.ShapeDtypeStruct((B, C, 2, n), jnp.float32),
        mesh=plsc.VectorSubcoreMesh(core_axis_name="c", subcore_axis_name="s"),
        compiler_params=pltpu.CompilerParams(needs_layout_passes=False),
        scratch_types=[
            pltpu.VMEM((2, m), jnp.float32),
            pltpu.VMEM((m + lanes,), jnp.int32),
            pltpu.VMEM((2, n), jnp.float32),
        ],
    )
    def k(x_hbm, idx_hbm, out_hbm, x_v, idx_v, out_v):
        wid = lax.axis_index("s") * nc + lax.axis_index("c")
        iota = lax.iota(jnp.int32, lanes)
        zeros_f = jnp.zeros((lanes,), jnp.float32)
        zeros_i = jnp.zeros((lanes,), jnp.int32)
        ones_i = jnp.ones((lanes,), jnp.int32)
        idx_v[pl.ds(m, lanes)] = jnp.full((lanes,), 2 ** 30, jnp.int32)

        def row_body(r, carry):
            row = wid * rows_per_w + r
            b = row // C
            c = row % C
            pltpu.sync_copy(x_hbm.at[b, c], x_v)
            pltpu.sync_copy(idx_hbm.at[b, c], idx_v.at[pl.ds(0, m)])

            def zero_body(i, cc):
                off = pl.multiple_of(i * lanes, lanes)
                out_v[0, pl.ds(off, lanes)] = zeros_f
                out_v[1, pl.ds(off, lanes)] = zeros_f
                return cc

            lax.fori_loop(0, n // lanes, zero_body, 0)

            def scat_body(i, cc):
                base = pl.multiple_of(i * lanes, lanes)
                d = idx_v[pl.ds(base, lanes)]
                d_next = plsc.load_gather(idx_v, [base + 1 + iota])
                keep = d != d_next
                reals = x_v[0, pl.ds(base, lanes)]
                imags = x_v[1, pl.ds(base, lanes)]
                plsc.store_scatter(out_v, [zeros_i, d], reals, mask=keep)
                plsc.store_scatter(out_v, [ones_i, d], imags, mask=keep)
                return cc

            lax.fori_loop(0, m // lanes, scat_body, 0)
            pltpu.sync_copy(out_v, out_hbm.at[b, c])
            return carry

        lax.fori_loop(0, rows_per_w, row_body, 0)

    return k


def kernel(x, indices):
    B, C, m, _ = x.shape
    n = _P * m
    xt = jnp.transpose(x, (0, 1, 3, 2))
    idx = indices.astype(jnp.int32)
    out_t = _make_unpool(B, C, m)(xt, idx)
    return jnp.transpose(out_t, (0, 1, 3, 2))


# speedup vs baseline: 5214.4363x; 1.8635x over previous
import functools

import jax
import jax.numpy as jnp
from jax import lax
from jax.experimental import pallas as pl
from jax.experimental.pallas import tpu as pltpu
from jax.experimental.pallas import tpu_sc as plsc

_P = 2


def _make_unpool(B, C, m):
    n = _P * m
    rows = B * C
    info = plsc.get_sparse_core_info()
    nc, ns, lanes = info.num_cores, info.num_subcores, info.num_lanes
    nw = nc * ns
    rows_per_w = rows // nw

    @functools.partial(
        pl.kernel,
        out_type=jax.ShapeDtypeStruct((B, C, 2, n), jnp.float32),
        mesh=plsc.VectorSubcoreMesh(core_axis_name="c", subcore_axis_name="s"),
        compiler_params=pltpu.CompilerParams(needs_layout_passes=False),
        scratch_types=[
            pltpu.VMEM((2, m), jnp.float32),
            pltpu.VMEM((2, m), jnp.float32),
            pltpu.VMEM((m + lanes,), jnp.int32),
            pltpu.VMEM((m + lanes,), jnp.int32),
            pltpu.VMEM((2, n), jnp.float32),
            pltpu.VMEM((2, n), jnp.float32),
            pltpu.SemaphoreType.DMA,
            pltpu.SemaphoreType.DMA,
            pltpu.SemaphoreType.DMA,
            pltpu.SemaphoreType.DMA,
        ],
    )
    def k(x_hbm, idx_hbm, out_hbm, x_v0, x_v1, i_v0, i_v1, o_v0, o_v1,
          si0, si1, so0, so1):
        wid = lax.axis_index("s") * nc + lax.axis_index("c")
        iota = lax.iota(jnp.int32, lanes)
        zeros_f = jnp.zeros((lanes,), jnp.float32)
        zeros_i = jnp.zeros((lanes,), jnp.int32)
        ones_i = jnp.ones((lanes,), jnp.int32)
        sentinel = jnp.full((lanes,), 2 ** 30, jnp.int32)
        i_v0[pl.ds(m, lanes)] = sentinel
        i_v1[pl.ds(m, lanes)] = sentinel

        bufs = [(x_v0, i_v0, o_v0, si0, so0), (x_v1, i_v1, o_v1, si1, so1)]
        base_row = wid * rows_per_w

        def bc(r):
            row = base_row + r
            return row // C, row % C

        def start_in(r):
            x_v, i_v, _, si, _ = bufs[r % 2]
            b, c = bc(r)
            hx = pltpu.async_copy(x_hbm.at[b, c], x_v, si)
            hi = pltpu.async_copy(idx_hbm.at[b, c], i_v.at[pl.ds(0, m)], si)
            return hx, hi

        def zero(o_v):
            def zero_body(i, cc):
                off = pl.multiple_of(i * lanes, lanes)
                o_v[0, pl.ds(off, lanes)] = zeros_f
                o_v[1, pl.ds(off, lanes)] = zeros_f
                return cc

            lax.fori_loop(0, n // lanes, zero_body, 0, unroll=8)

        def scatter(x_v, i_v, o_v):
            def scat_body(i, cc):
                base = pl.multiple_of(i * lanes, lanes)
                d = i_v[pl.ds(base, lanes)]
                d_next = plsc.load_gather(i_v, [base + 1 + iota])
                keep = d != d_next
                reals = x_v[0, pl.ds(base, lanes)]
                imags = x_v[1, pl.ds(base, lanes)]
                plsc.store_scatter(o_v, [zeros_i, d], reals, mask=keep)
                plsc.store_scatter(o_v, [ones_i, d], imags, mask=keep)
                return cc

            lax.fori_loop(0, m // lanes, scat_body, 0, unroll=4)

        in_h = {0: start_in(0)}
        out_h = {}
        for r in range(rows_per_w):
            x_v, i_v, o_v, _, so = bufs[r % 2]
            if r + 1 < rows_per_w:
                in_h[r + 1] = start_in(r + 1)
            if r >= 2:
                out_h[r - 2].wait()
            zero(o_v)
            hx, hi = in_h.pop(r)
            hx.wait()
            hi.wait()
            scatter(x_v, i_v, o_v)
            b, c = bc(r)
            out_h[r] = pltpu.async_copy(o_v, out_hbm.at[b, c], so)
        out_h[rows_per_w - 2].wait()
        out_h[rows_per_w - 1].wait()

    return k


def kernel(x, indices):
    B, C, m, _ = x.shape
    xt = jnp.transpose(x, (0, 1, 3, 2))
    idx = indices.astype(jnp.int32)
    out_t = _make_unpool(B, C, m)(xt, idx)
    return jnp.transpose(out_t, (0, 1, 3, 2))


# speedup vs baseline: 9524.0457x; 1.8265x over previous
import functools

import jax
import jax.numpy as jnp
from jax import lax
from jax.experimental import pallas as pl
from jax.experimental.pallas import tpu as pltpu
from jax.experimental.pallas import tpu_sc as plsc

_P = 2


def _make_unpool(B, C, m):
    n = _P * m
    rows = B * C
    info = plsc.get_sparse_core_info()
    nc, ns, lanes = info.num_cores, info.num_subcores, info.num_lanes
    nw = nc * ns
    rows_per_w = rows // nw

    @functools.partial(
        pl.kernel,
        out_type=jax.ShapeDtypeStruct((B, C, 2, n), jnp.float32),
        mesh=plsc.VectorSubcoreMesh(core_axis_name="c", subcore_axis_name="s"),
        compiler_params=pltpu.CompilerParams(needs_layout_passes=False),
        scratch_types=[
            pltpu.VMEM((2, m), jnp.float32),
            pltpu.VMEM((2, m), jnp.float32),
            pltpu.VMEM((m + lanes,), jnp.int32),
            pltpu.VMEM((m + lanes,), jnp.int32),
            pltpu.VMEM((2, n), jnp.float32),
            pltpu.VMEM((2, n), jnp.float32),
            pltpu.SemaphoreType.DMA,
            pltpu.SemaphoreType.DMA,
            pltpu.SemaphoreType.DMA,
            pltpu.SemaphoreType.DMA,
        ],
    )
    def k(x_hbm, idx_hbm, out_hbm, x_v0, x_v1, i_v0, i_v1, o_v0, o_v1,
          si0, si1, so0, so1):
        wid = lax.axis_index("s") * nc + lax.axis_index("c")
        iota = lax.iota(jnp.int32, lanes)
        zeros_f = jnp.zeros((lanes,), jnp.float32)
        zeros_i = jnp.zeros((lanes,), jnp.int32)
        ones_i = jnp.ones((lanes,), jnp.int32)
        sentinel = jnp.full((lanes,), 2 ** 30, jnp.int32)
        i_v0[pl.ds(m, lanes)] = sentinel
        i_v1[pl.ds(m, lanes)] = sentinel

        bufs = [(x_v0, i_v0, o_v0, si0, so0), (x_v1, i_v1, o_v1, si1, so1)]
        base_row = wid * rows_per_w

        def bc(r):
            row = base_row + r
            return row // C, row % C

        def start_in(r):
            x_v, i_v, _, si, _ = bufs[r % 2]
            b, c = bc(r)
            hx = pltpu.async_copy(x_hbm.at[b, c], x_v, si)
            hi = pltpu.async_copy(idx_hbm.at[b, c], i_v.at[pl.ds(0, m)], si)
            return hx, hi

        def zero(o_v):
            @plsc.parallel_loop(0, n, step=lanes, unroll=8)
            def zero_body(i):
                off = pl.multiple_of(i, lanes)
                o_v[0, pl.ds(off, lanes)] = zeros_f
                o_v[1, pl.ds(off, lanes)] = zeros_f

        def scatter(x_v, i_v, o_v):
            @plsc.parallel_loop(0, m, step=lanes, unroll=8)
            def scat_body(i):
                base = pl.multiple_of(i, lanes)
                d = i_v[pl.ds(base, lanes)]
                d_next = plsc.load_gather(i_v, [base + 1 + iota])
                keep = d != d_next
                reals = x_v[0, pl.ds(base, lanes)]
                imags = x_v[1, pl.ds(base, lanes)]
                plsc.store_scatter(o_v, [zeros_i, d], reals, mask=keep)
                plsc.store_scatter(o_v, [ones_i, d], imags, mask=keep)

        in_h = {0: start_in(0)}
        out_h = {}
        for r in range(rows_per_w):
            x_v, i_v, o_v, _, so = bufs[r % 2]
            if r + 1 < rows_per_w:
                in_h[r + 1] = start_in(r + 1)
            if r >= 2:
                out_h[r - 2].wait()
            zero(o_v)
            hx, hi = in_h.pop(r)
            hx.wait()
            hi.wait()
            scatter(x_v, i_v, o_v)
            b, c = bc(r)
            out_h[r] = pltpu.async_copy(o_v, out_hbm.at[b, c], so)
        out_h[rows_per_w - 2].wait()
        out_h[rows_per_w - 1].wait()

    return k


def kernel(x, indices):
    B, C, m, _ = x.shape
    xt = jnp.transpose(x, (0, 1, 3, 2))
    idx = indices.astype(jnp.int32)
    out_t = _make_unpool(B, C, m)(xt, idx)
    return jnp.transpose(out_t, (0, 1, 3, 2))
